# baseline (speedup 1.0000x reference)
import functools
import math

import jax
import jax.numpy as jnp
from jax import lax
from jax.experimental import pallas as pl
from jax.experimental.pallas import tpu as pltpu
from jax.experimental.pallas import tpu_sc as plsc

D_MODEL = 64
_SCALE = math.sqrt(D_MODEL)
_CHUNK = 512


@functools.lru_cache(maxsize=None)
def _make_sc_kernel(n_rows: int):
    info = plsc.get_sparse_core_info()
    num_workers = info.num_cores * info.num_subcores
    rows_per_worker = n_rows // num_workers
    assert rows_per_worker * num_workers == n_rows
    n_chunks = rows_per_worker // _CHUNK
    assert n_chunks * _CHUNK == rows_per_worker

    mesh = plsc.VectorSubcoreMesh(core_axis_name="c", subcore_axis_name="s")

    @functools.partial(
        pl.kernel,
        mesh=mesh,
        out_type=jax.ShapeDtypeStruct((n_rows, D_MODEL), jnp.float32),
        scratch_types=[
            pltpu.VMEM((_CHUNK,), jnp.int32),
            pltpu.VMEM((_CHUNK, D_MODEL), jnp.float32),
            pltpu.SemaphoreType.DMA,
        ],
        compiler_params=pltpu.CompilerParams(use_tc_tiling_on_sc=False),
    )
    def sc_kernel(x_hbm, lut_hbm, out_hbm, idx_v, rows_v, sem):
        wid = lax.axis_index("s") * info.num_cores + lax.axis_index("c")
        base = wid * rows_per_worker

        def chunk_body(ci, carry):
            off = base + ci * _CHUNK
            pltpu.sync_copy(x_hbm.at[pl.ds(off, _CHUNK)], idx_v)
            pltpu.async_copy(lut_hbm.at[idx_v], rows_v, sem).wait()

            def scale_row(r, c2):
                for j in range(D_MODEL // 16):
                    sl = (r, pl.ds(j * 16, 16))
                    rows_v[sl] = rows_v[sl] * _SCALE
                return c2

            lax.fori_loop(0, _CHUNK, scale_row, 0)
            pltpu.sync_copy(rows_v, out_hbm.at[pl.ds(off, _CHUNK)])
            return carry

        lax.fori_loop(0, n_chunks, chunk_body, 0)

    return sc_kernel


def kernel(x, lut):
    b, s = x.shape
    flat = x.reshape(b * s)
    out = _make_sc_kernel(b * s)(flat, lut)
    return out.reshape(b, s, D_MODEL)


# speedup vs baseline: 1.1397x; 1.1397x over previous
import functools
import math

import jax
import jax.numpy as jnp
from jax import lax
from jax.experimental import pallas as pl
from jax.experimental.pallas import tpu as pltpu
from jax.experimental.pallas import tpu_sc as plsc

D_MODEL = 64
_SCALE = math.sqrt(D_MODEL)
_CHUNK = 400
_NBUF = 4
_LOOK = 2


def _scale_chunk(rows_v):
    rows_per_iter = 8

    def body(rr, carry):
        r0 = rr * rows_per_iter
        for dr in range(rows_per_iter):
            for j in range(D_MODEL // 16):
                sl = (r0 + dr, pl.ds(j * 16, 16))
                rows_v[sl] = rows_v[sl] * _SCALE
        return carry

    lax.fori_loop(0, _CHUNK // rows_per_iter, body, 0)


@functools.lru_cache(maxsize=None)
def _make_sc_kernel(n_rows: int):
    info = plsc.get_sparse_core_info()
    num_workers = info.num_cores * info.num_subcores
    rows_per_worker = n_rows // num_workers
    assert rows_per_worker * num_workers == n_rows
    n_chunks = rows_per_worker // _CHUNK
    assert n_chunks * _CHUNK == rows_per_worker
    assert n_chunks % _NBUF == 0 and n_chunks >= 2 * _NBUF

    mesh = plsc.VectorSubcoreMesh(core_axis_name="c", subcore_axis_name="s")

    @functools.partial(
        pl.kernel,
        mesh=mesh,
        out_type=jax.ShapeDtypeStruct((n_rows, D_MODEL), jnp.float32),
        scratch_types=(
            [pltpu.VMEM((_CHUNK,), jnp.int32) for _ in range(_NBUF)]
            + [pltpu.VMEM((_CHUNK, D_MODEL), jnp.float32) for _ in range(_NBUF)]
            + [pltpu.SemaphoreType.DMA for _ in range(2 * _NBUF)]
        ),
        compiler_params=pltpu.CompilerParams(use_tc_tiling_on_sc=False),
    )
    def sc_kernel(x_hbm, lut_hbm, out_hbm, *scratch):
        idx_bufs = scratch[:_NBUF]
        rows_bufs = scratch[_NBUF : 2 * _NBUF]
        gsems = scratch[2 * _NBUF : 3 * _NBUF]
        osems = scratch[3 * _NBUF : 4 * _NBUF]

        wid = lax.axis_index("s") * info.num_cores + lax.axis_index("c")
        base = wid * rows_per_worker

        def issue_gather(chunk, b):
            off = base + chunk * _CHUNK
            pltpu.sync_copy(x_hbm.at[pl.ds(off, _CHUNK)], idx_bufs[b])
            pltpu.async_copy(lut_hbm.at[idx_bufs[b]], rows_bufs[b], gsems[b])

        def wait_gather(b):
            pltpu.make_async_copy(
                lut_hbm.at[idx_bufs[b]], rows_bufs[b], gsems[b]
            ).wait()

        def issue_out(chunk, b):
            off = base + chunk * _CHUNK
            pltpu.async_copy(
                rows_bufs[b], out_hbm.at[pl.ds(off, _CHUNK)], osems[b]
            )

        def wait_out(b):
            pltpu.make_async_copy(
                rows_bufs[b], out_hbm.at[pl.ds(0, _CHUNK)], osems[b]
            ).wait()

        for i in range(_LOOK):
            issue_gather(i, i)

        def outer(it, carry):
            for b in range(_NBUF):
                i = it * _NBUF + b
                j = i + _LOOK
                bj = (b + _LOOK) % _NBUF

                @pl.when(jnp.logical_and(j >= _NBUF, j < n_chunks))
                def _():
                    wait_out(bj)

                @pl.when(j < n_chunks)
                def _():
                    issue_gather(j, bj)

                wait_gather(b)
                _scale_chunk(rows_bufs[b])
                issue_out(i, b)
            return carry

        lax.fori_loop(0, n_chunks // _NBUF, outer, 0)

        for b in range(_NBUF):
            wait_out(b)

    return sc_kernel


def kernel(x, lut):
    b, s = x.shape
    flat = x.reshape(b * s)
    out = _make_sc_kernel(b * s)(flat, lut)
    return out.reshape(b, s, D_MODEL)
